# Initial kernel scaffold; baseline (speedup 1.0000x reference)
import jax
import jax.numpy as jnp
from jax.experimental import pallas as pl


def kernel(x, edge_index, edge_attr, W_enc, b_enc, emb, eps1, W1a, b1a, W1b, b1b, W1c, b1c, eps2, W2a, b2a, W2b, b2b, W2c, b2c):
    raise NotImplementedError("write your pallas kernel here")


# speedup vs baseline: 1.3185x; 1.3185x over previous
import functools

import jax
import jax.numpy as jnp
from jax import lax
from jax.experimental import pallas as pl
from jax.experimental.pallas import tpu as pltpu
from jax.experimental.pallas import tpu_sc as plsc

N = 10000
E = 320000
D = 128

NC = 2
NS = 16
NW = NC * NS

EPT = E // NW
C = 80
NCHUNK = EPT // C
RPT = 624
TAIL = N - NS * RPT
ZROWS = 104


def _sc_aggregate(h, src, dst, attr, emb):
    mesh = plsc.VectorSubcoreMesh(core_axis_name="c", subcore_axis_name="s")

    @functools.partial(
        pl.kernel,
        mesh=mesh,
        out_type=jax.ShapeDtypeStruct((NC, N, D), jnp.float32),
        scratch_types=[
            pltpu.VMEM((C,), jnp.int32),
            pltpu.VMEM((C,), jnp.int32),
            pltpu.VMEM((1, C), jnp.int32),
            pltpu.VMEM((C, D), jnp.float32),
            pltpu.VMEM((C, D), jnp.float32),
            pltpu.VMEM((ZROWS, D), jnp.float32),
            pltpu.VMEM_SHARED((N, D), jnp.float32),
            pltpu.SemaphoreType.DMA,
            pltpu.SemaphoreType.DMA,
        ],
    )
    def agg(h_hbm, src_hbm, dst_hbm, attr_hbm, emb_hbm, out_hbm,
            sidx, aidx, didx, hrow, erow, zbuf, accum, sem1, sem2):
        c = lax.axis_index("c")
        s = lax.axis_index("s")

        zero = jnp.zeros((16,), jnp.float32)

        def zb_body(i, carry):
            for j in range(8):
                zbuf[i, pl.ds(j * 16, 16)] = zero
            return carry

        lax.fori_loop(0, ZROWS, zb_body, 0)
        for k in range(RPT // ZROWS):
            pltpu.sync_copy(zbuf, accum.at[pl.ds(s * RPT + k * ZROWS, ZROWS)])

        @pl.when(s == NS - 1)
        def _zero_tail():
            pltpu.sync_copy(zbuf.at[pl.ds(0, TAIL)],
                            accum.at[pl.ds(NS * RPT, TAIL)])

        plsc.subcore_barrier()

        base = (c * NS + s) * EPT

        def chunk_body(i, carry):
            eb = base + i * C
            pltpu.sync_copy(src_hbm.at[pl.ds(eb, C)], sidx)
            pltpu.sync_copy(attr_hbm.at[pl.ds(eb, C)], aidx)
            pltpu.sync_copy(dst_hbm.at[pl.ds(eb, C)], didx.at[0])
            cp1 = pltpu.async_copy(h_hbm.at[sidx], hrow, sem1)
            cp2 = pltpu.async_copy(emb_hbm.at[aidx], erow, sem2)
            cp1.wait()
            cp2.wait()

            def edge_body(e, ecarry):
                for j in range(8):
                    sl = pl.ds(j * 16, 16)
                    hrow[e, sl] = jnp.maximum(hrow[e, sl] + erow[e, sl], 0.0)
                return ecarry

            lax.fori_loop(0, C, edge_body, 0)
            pltpu.sync_copy(hrow, accum.at[didx.at[0]], add=True)
            return carry

        lax.fori_loop(0, NCHUNK, chunk_body, 0)
        plsc.subcore_barrier()

        pltpu.sync_copy(accum.at[pl.ds(s * RPT, RPT)],
                        out_hbm.at[c, pl.ds(s * RPT, RPT)])

        @pl.when(s == NS - 1)
        def _copy_tail():
            pltpu.sync_copy(accum.at[pl.ds(NS * RPT, TAIL)],
                            out_hbm.at[c, pl.ds(NS * RPT, TAIL)])

    return agg(h, src, dst, attr, emb)


BLK = 1000


def _enc_body(x_ref, w_ref, b_ref, o_ref):
    z = jnp.dot(x_ref[...], w_ref[...], preferred_element_type=jnp.float32)
    o_ref[...] = jnp.maximum(z + b_ref[...], 0.0)


def _encode(x, w, b):
    return pl.pallas_call(
        _enc_body,
        grid=(N // BLK,),
        in_specs=[
            pl.BlockSpec((BLK, D), lambda i: (i, 0)),
            pl.BlockSpec((D, D), lambda i: (0, 0)),
            pl.BlockSpec((1, D), lambda i: (0, 0)),
        ],
        out_specs=pl.BlockSpec((BLK, D), lambda i: (i, 0)),
        out_shape=jax.ShapeDtypeStruct((N, D), jnp.float32),
    )(x, w, b.reshape(1, D))


def _mlp_body(h_ref, p_ref, scale_ref, wa_ref, ba_ref, wb_ref, bb_ref,
              wc_ref, bc_ref, o_ref):
    z = h_ref[...] * scale_ref[...] + p_ref[0] + p_ref[1]
    z = jnp.dot(z, wa_ref[...], preferred_element_type=jnp.float32)
    z = jnp.maximum(z + ba_ref[...], 0.0)
    z = jnp.dot(z, wb_ref[...], preferred_element_type=jnp.float32)
    z = jnp.maximum(z + bb_ref[...], 0.0)
    z = jnp.dot(z, wc_ref[...], preferred_element_type=jnp.float32)
    o_ref[...] = jnp.maximum(z + bc_ref[...], 0.0)


def _mlp(h, partials, eps, wa, ba, wb, bb, wc, bc):
    scale = (1.0 + eps).reshape(1, 1)
    wspec = pl.BlockSpec((D, D), lambda i: (0, 0))
    bspec = pl.BlockSpec((1, D), lambda i: (0, 0))
    return pl.pallas_call(
        _mlp_body,
        grid=(N // BLK,),
        in_specs=[
            pl.BlockSpec((BLK, D), lambda i: (i, 0)),
            pl.BlockSpec((NC, BLK, D), lambda i: (0, i, 0)),
            pl.BlockSpec((1, 1), lambda i: (0, 0)),
            wspec, bspec, wspec, bspec, wspec, bspec,
        ],
        out_specs=pl.BlockSpec((BLK, D), lambda i: (i, 0)),
        out_shape=jax.ShapeDtypeStruct((N, D), jnp.float32),
    )(h, partials, scale, wa, ba.reshape(1, D), wb, bb.reshape(1, D),
      wc, bc.reshape(1, D))


def kernel(x, edge_index, edge_attr, W_enc, b_enc, emb, eps1,
           W1a, b1a, W1b, b1b, W1c, b1c, eps2, W2a, b2a, W2b, b2b, W2c, b2c):
    src = edge_index[0]
    dst = edge_index[1]

    h = _encode(x, W_enc, b_enc)
    p1 = _sc_aggregate(h, src, dst, edge_attr, emb)
    h = _mlp(h, p1, eps1, W1a, b1a, W1b, b1b, W1c, b1c)
    p2 = _sc_aggregate(h, src, dst, edge_attr, emb)
    h = _mlp(h, p2, eps2, W2a, b2a, W2b, b2b, W2c, b2c)
    return h
